# Initial kernel scaffold; baseline (speedup 1.0000x reference)
import jax
import jax.numpy as jnp
from jax.experimental import pallas as pl


def kernel(x, edge_index, W1f_l, b1f, W1f_r, W1b_l, b1b, W1b_r, bn1_g, bn1_b, W2f_l, b2f, W2f_r, W2b_l, b2b, W2b_r, bn2_g, bn2_b, Wp, bp):
    raise NotImplementedError("write your pallas kernel here")


# speedup vs baseline: 2.4026x; 2.4026x over previous
import jax
import jax.numpy as jnp
from jax import lax
from jax.experimental import pallas as pl
from jax.experimental.pallas import tpu as pltpu
from jax.experimental.pallas import tpu_sc as plsc

N = 10000
E = 320000
D = 128
HID = 256
EPS = 1e-5

NC = 2
NS = 16
CH = 128
SB = 16
NBLK = 10
EPT = NBLK * SB * CH
EPC = NS * EPT
NP = 10240
RPT = NP // NS
ZR = 32

_HIGH = lax.Precision.HIGHEST


def _dot(a, b):
    return jnp.dot(a, b, preferred_element_type=jnp.float32, precision=_HIGH)


def _mesh():
    return plsc.VectorSubcoreMesh(core_axis_name="c", subcore_axis_name="s",
                                  num_cores=NC, num_subcores=NS)


def _aggregate(ytab, gidx, sidx):
    def body(ytab_ref, gidx_ref, sidx_ref, agg_out, gslab, sslab, rows,
             zrow, agg_sh, sem):
        c = lax.axis_index("c")
        s = lax.axis_index("s")

        z16 = jnp.zeros((16,), jnp.float32)
        for i in range(ZR):
            for k in range(D // 16):
                zrow[i, pl.ds(k * 16, 16)] = z16

        base = s * RPT
        for t in range(RPT // ZR):
            pltpu.sync_copy(zrow, agg_sh.at[pl.ds(base + t * ZR, ZR)])
        plsc.subcore_barrier()

        def blk_step(blk, carry):
            pltpu.sync_copy(gidx_ref.at[c, s, blk], gslab)
            pltpu.sync_copy(sidx_ref.at[c, s, blk], sslab)
            for j in range(SB):
                pltpu.async_copy(ytab_ref.at[gslab.at[j]], rows, sem).wait()
                pltpu.sync_copy(rows, agg_sh.at[sslab.at[j]], add=True)
            return carry

        lax.fori_loop(0, NBLK, blk_step, 0)
        plsc.subcore_barrier()
        pltpu.sync_copy(agg_sh.at[pl.ds(base, RPT)],
                        agg_out.at[c, pl.ds(base, RPT)])

    return pl.kernel(
        body,
        out_type=jax.ShapeDtypeStruct((NC, NP, D), jnp.float32),
        mesh=_mesh(),
        scratch_types=[
            pltpu.VMEM((SB, CH), jnp.int32),
            pltpu.VMEM((SB, CH), jnp.int32),
            pltpu.VMEM((CH, D), jnp.float32),
            pltpu.VMEM((ZR, D), jnp.float32),
            pltpu.VMEM_SHARED((NP, D), jnp.float32),
            pltpu.SemaphoreType.DMA,
        ],
    )(ytab, gidx, sidx)


def _counts(sidx):
    def body(sidx_ref, cnt_out, sslab, ones, zcnt, cnt_sh):
        c = lax.axis_index("c")
        s = lax.axis_index("s")

        z16 = jnp.zeros((16,), jnp.float32)
        o16 = jnp.ones((16,), jnp.float32)
        for i in range(ZR):
            zcnt[i, :] = z16
        for i in range(CH):
            ones[i, :] = o16

        base = s * RPT
        for t in range(RPT // ZR):
            pltpu.sync_copy(zcnt, cnt_sh.at[pl.ds(base + t * ZR, ZR)])
        plsc.subcore_barrier()

        def blk_step(blk, carry):
            pltpu.sync_copy(sidx_ref.at[c, s, blk], sslab)
            for j in range(SB):
                pltpu.sync_copy(ones, cnt_sh.at[sslab.at[j]], add=True)
            return carry

        lax.fori_loop(0, NBLK, blk_step, 0)
        plsc.subcore_barrier()
        pltpu.sync_copy(cnt_sh.at[pl.ds(base, RPT)],
                        cnt_out.at[c, pl.ds(base, RPT)])

    return pl.kernel(
        body,
        out_type=jax.ShapeDtypeStruct((NC, NP, 16), jnp.float32),
        mesh=_mesh(),
        scratch_types=[
            pltpu.VMEM((SB, CH), jnp.int32),
            pltpu.VMEM((CH, 16), jnp.float32),
            pltpu.VMEM((ZR, 16), jnp.float32),
            pltpu.VMEM_SHARED((NP, 16), jnp.float32),
        ],
    )(sidx)


_NB = 10
_BLK = N // _NB


def _tc_pre(x, wfl, wbl, wfr, wbr):
    def body(x_ref, wfl_ref, wbl_ref, wfr_ref, wbr_ref, y_ref, r_ref):
        xb = x_ref[...]
        y_ref[0] = _dot(xb, wfl_ref[...])
        y_ref[1] = _dot(xb, wbl_ref[...])
        r_ref[0] = _dot(xb, wfr_ref[...])
        r_ref[1] = _dot(xb, wbr_ref[...])

    w_spec = pl.BlockSpec((D, D), lambda i: (0, 0))
    return pl.pallas_call(
        body,
        grid=(_NB,),
        in_specs=[pl.BlockSpec((_BLK, D), lambda i: (i, 0)),
                  w_spec, w_spec, w_spec, w_spec],
        out_specs=[pl.BlockSpec((2, _BLK, D), lambda i: (0, i, 0)),
                   pl.BlockSpec((2, _BLK, D), lambda i: (0, i, 0))],
        out_shape=[jax.ShapeDtypeStruct((2, N, D), jnp.float32),
                   jax.ShapeDtypeStruct((2, N, D), jnp.float32)],
    )(x, wfl, wbl, wfr, wbr)


def _tc_stats(agg, cnt, rtab, bf, bb):
    def body(agg_ref, cnt_ref, rtab_ref, bf_ref, bb_ref, h_ref, s_ref, q_ref):
        i = pl.program_id(0)
        cin = jnp.maximum(cnt_ref[0, :, 0:1], 1.0)
        cout = jnp.maximum(cnt_ref[1, :, 0:1], 1.0)
        hf = agg_ref[0] / cin + bf_ref[...] + rtab_ref[0]
        hb = agg_ref[1] / cout + bb_ref[...] + rtab_ref[1]
        h = jnp.concatenate([hf, hb], axis=1)
        h_ref[...] = h

        @pl.when(i == 0)
        def _():
            s_ref[...] = jnp.zeros_like(s_ref)
            q_ref[...] = jnp.zeros_like(q_ref)

        s_ref[...] += jnp.sum(h, axis=0, keepdims=True)
        q_ref[...] += jnp.sum(h * h, axis=0, keepdims=True)

    b_spec = pl.BlockSpec((1, D), lambda i: (0, 0))
    stat_spec = pl.BlockSpec((1, HID), lambda i: (0, 0))
    return pl.pallas_call(
        body,
        grid=(_NB,),
        in_specs=[pl.BlockSpec((2, _BLK, D), lambda i: (0, i, 0)),
                  pl.BlockSpec((2, _BLK, D), lambda i: (0, i, 0)),
                  pl.BlockSpec((2, _BLK, D), lambda i: (0, i, 0)),
                  b_spec, b_spec],
        out_specs=[pl.BlockSpec((_BLK, HID), lambda i: (i, 0)),
                   stat_spec, stat_spec],
        out_shape=[jax.ShapeDtypeStruct((N, HID), jnp.float32),
                   jax.ShapeDtypeStruct((1, HID), jnp.float32),
                   jax.ShapeDtypeStruct((1, HID), jnp.float32)],
    )(agg, cnt, rtab, bf, bb)


def _bn_relu(h, s, q, g, b):
    m = s / N
    v = q / N - m * m
    return jnp.maximum((h - m) * lax.rsqrt(v + EPS) * g + b, 0.0)


def _tc_apply_mid(h, s, q, g, b, wfl, wbl, wfr, wbr):
    def body(h_ref, s_ref, q_ref, g_ref, b_ref,
             wfl_ref, wbl_ref, wfr_ref, wbr_ref, y2_ref, r2_ref):
        hn = _bn_relu(h_ref[...], s_ref[...], q_ref[...],
                      g_ref[...], b_ref[...])
        y2_ref[0] = _dot(hn, wfl_ref[...])
        y2_ref[1] = _dot(hn, wbl_ref[...])
        r2_ref[0] = _dot(hn, wfr_ref[...])
        r2_ref[1] = _dot(hn, wbr_ref[...])

    stat_spec = pl.BlockSpec((1, HID), lambda i: (0, 0))
    w_spec = pl.BlockSpec((HID, D), lambda i: (0, 0))
    return pl.pallas_call(
        body,
        grid=(_NB,),
        in_specs=[pl.BlockSpec((_BLK, HID), lambda i: (i, 0)),
                  stat_spec, stat_spec, stat_spec, stat_spec,
                  w_spec, w_spec, w_spec, w_spec],
        out_specs=[pl.BlockSpec((2, _BLK, D), lambda i: (0, i, 0)),
                   pl.BlockSpec((2, _BLK, D), lambda i: (0, i, 0))],
        out_shape=[jax.ShapeDtypeStruct((2, N, D), jnp.float32),
                   jax.ShapeDtypeStruct((2, N, D), jnp.float32)],
    )(h, s, q, g, b, wfl, wbl, wfr, wbr)


def _tc_apply_post(h, s, q, g, b, wp, bp):
    def body(h_ref, s_ref, q_ref, g_ref, b_ref, wp_ref, bp_ref,
             embs_ref, genc_ref):
        i = pl.program_id(0)
        embs = _bn_relu(h_ref[...], s_ref[...], q_ref[...],
                        g_ref[...], b_ref[...])
        embs_ref[...] = embs
        proj = jnp.maximum(_dot(embs, wp_ref[...]) + bp_ref[...], 0.0)
        pmax = jnp.max(proj, axis=0, keepdims=True)

        @pl.when(i == 0)
        def _():
            genc_ref[...] = jnp.zeros_like(genc_ref)

        genc_ref[...] = jnp.maximum(genc_ref[...], pmax)

    stat_spec = pl.BlockSpec((1, HID), lambda i: (0, 0))
    return pl.pallas_call(
        body,
        grid=(_NB,),
        in_specs=[pl.BlockSpec((_BLK, HID), lambda i: (i, 0)),
                  stat_spec, stat_spec, stat_spec, stat_spec,
                  pl.BlockSpec((HID, HID), lambda i: (0, 0)),
                  stat_spec],
        out_specs=[pl.BlockSpec((_BLK, HID), lambda i: (i, 0)),
                   stat_spec],
        out_shape=[jax.ShapeDtypeStruct((N, HID), jnp.float32),
                   jax.ShapeDtypeStruct((1, HID), jnp.float32)],
    )(h, s, q, g, b, wp, bp)


def kernel(x, edge_index, W1f_l, b1f, W1f_r, W1b_l, b1b, W1b_r, bn1_g, bn1_b,
           W2f_l, b2f, W2f_r, W2b_l, b2b, W2b_r, bn2_g, bn2_b, Wp, bp):
    src = edge_index[0]
    dst = edge_index[1]
    pad = EPC - E
    zpad = jnp.zeros((pad,), jnp.int32)
    tpad = jnp.full((pad,), N, jnp.int32)
    gidx = jnp.stack([jnp.concatenate([src, zpad]),
                      jnp.concatenate([dst + N, zpad])])
    sidx = jnp.stack([jnp.concatenate([dst, tpad]),
                      jnp.concatenate([src, tpad])])
    gidx = gidx.reshape(NC, NS, NBLK, SB, CH)
    sidx = sidx.reshape(NC, NS, NBLK, SB, CH)

    ones_tab = jnp.ones((NC * N, D), jnp.float32)
    cnt = _aggregate(ones_tab, gidx, sidx)
    ytab1, rtab1 = _tc_pre(x, W1f_l, W1b_l, W1f_r, W1b_r)
    agg1 = _aggregate(ytab1.reshape(NC * N, D), gidx, sidx)
    h1, s1, q1 = _tc_stats(agg1, cnt, rtab1,
                           b1f.reshape(1, D), b1b.reshape(1, D))
    ytab2, rtab2 = _tc_apply_mid(h1, s1, q1,
                                 bn1_g.reshape(1, HID), bn1_b.reshape(1, HID),
                                 W2f_l, W2b_l, W2f_r, W2b_r)
    agg2 = _aggregate(ytab2.reshape(NC * N, D), gidx, sidx)
    h2, s2, q2 = _tc_stats(agg2, cnt, rtab2,
                           b2f.reshape(1, D), b2b.reshape(1, D))
    node_embs, genc = _tc_apply_post(h2, s2, q2,
                                     bn2_g.reshape(1, HID),
                                     bn2_b.reshape(1, HID),
                                     Wp, bp.reshape(1, HID))
    return (node_embs, genc.reshape(HID))


# speedup vs baseline: 2.6217x; 1.0912x over previous
import jax
import jax.numpy as jnp
from jax import lax
from jax.experimental import pallas as pl
from jax.experimental.pallas import tpu as pltpu
from jax.experimental.pallas import tpu_sc as plsc

N = 10000
E = 320000
D = 128
HID = 256
EPS = 1e-5

NC = 2
NS = 16
CH = 128
SB = 16
NBLK = 10
EPT = NBLK * SB * CH
EPC = NS * EPT
NP = 10240
RPT = NP // NS
ZR = 32

_HIGH = lax.Precision.HIGHEST


def _dot(a, b):
    return jnp.dot(a, b, preferred_element_type=jnp.float32, precision=_HIGH)


def _mesh():
    return plsc.VectorSubcoreMesh(core_axis_name="c", subcore_axis_name="s",
                                  num_cores=NC, num_subcores=NS)


def _aggregate(ytab, gidx, sidx):
    def body(ytab_ref, gidx_ref, sidx_ref, agg_out, gslab, sslab, rows0,
             rows1, zrow, agg_sh, gsem0, gsem1, ssem0, ssem1):
        c = lax.axis_index("c")
        s = lax.axis_index("s")

        z16 = jnp.zeros((16,), jnp.float32)
        for i in range(ZR):
            for k in range(D // 16):
                zrow[i, pl.ds(k * 16, 16)] = z16

        base = s * RPT
        for t in range(RPT // ZR):
            pltpu.sync_copy(zrow, agg_sh.at[pl.ds(base + t * ZR, ZR)])
        plsc.subcore_barrier()

        bufs = (rows0, rows1)
        gsems = (gsem0, gsem1)
        ssems = (ssem0, ssem1)

        def blk_step(blk, carry):
            pltpu.sync_copy(gidx_ref.at[c, s, blk], gslab)
            pltpu.sync_copy(sidx_ref.at[c, s, blk], sslab)
            gd = [None] * SB
            sd = [None] * SB
            gd[0] = pltpu.async_copy(ytab_ref.at[gslab.at[0]], rows0, gsem0)
            for j in range(SB):
                if j >= 1:
                    sd[j - 1].wait()
                if j + 1 < SB:
                    gd[j + 1] = pltpu.async_copy(
                        ytab_ref.at[gslab.at[j + 1]],
                        bufs[(j + 1) % 2], gsems[(j + 1) % 2])
                gd[j].wait()
                sd[j] = pltpu.async_copy(bufs[j % 2],
                                         agg_sh.at[sslab.at[j]],
                                         ssems[j % 2], add=True)
            sd[SB - 1].wait()
            return carry

        lax.fori_loop(0, NBLK, blk_step, 0)
        plsc.subcore_barrier()
        pltpu.sync_copy(agg_sh.at[pl.ds(base, RPT)],
                        agg_out.at[c, pl.ds(base, RPT)])

    return pl.kernel(
        body,
        out_type=jax.ShapeDtypeStruct((NC, NP, D), jnp.float32),
        mesh=_mesh(),
        scratch_types=[
            pltpu.VMEM((SB, CH), jnp.int32),
            pltpu.VMEM((SB, CH), jnp.int32),
            pltpu.VMEM((CH, D), jnp.float32),
            pltpu.VMEM((CH, D), jnp.float32),
            pltpu.VMEM((ZR, D), jnp.float32),
            pltpu.VMEM_SHARED((NP, D), jnp.float32),
            pltpu.SemaphoreType.DMA,
            pltpu.SemaphoreType.DMA,
            pltpu.SemaphoreType.DMA,
            pltpu.SemaphoreType.DMA,
        ],
    )(ytab, gidx, sidx)


def _counts(sidx):
    def body(sidx_ref, cnt_out, sslab, ones, zcnt, cnt_sh):
        c = lax.axis_index("c")
        s = lax.axis_index("s")

        z16 = jnp.zeros((16,), jnp.float32)
        o16 = jnp.ones((16,), jnp.float32)
        for i in range(ZR):
            zcnt[i, :] = z16
        for i in range(CH):
            ones[i, :] = o16

        base = s * RPT
        for t in range(RPT // ZR):
            pltpu.sync_copy(zcnt, cnt_sh.at[pl.ds(base + t * ZR, ZR)])
        plsc.subcore_barrier()

        def blk_step(blk, carry):
            pltpu.sync_copy(sidx_ref.at[c, s, blk], sslab)
            for j in range(SB):
                pltpu.sync_copy(ones, cnt_sh.at[sslab.at[j]], add=True)
            return carry

        lax.fori_loop(0, NBLK, blk_step, 0)
        plsc.subcore_barrier()
        pltpu.sync_copy(cnt_sh.at[pl.ds(base, RPT)],
                        cnt_out.at[c, pl.ds(base, RPT)])

    return pl.kernel(
        body,
        out_type=jax.ShapeDtypeStruct((NC, NP, 16), jnp.float32),
        mesh=_mesh(),
        scratch_types=[
            pltpu.VMEM((SB, CH), jnp.int32),
            pltpu.VMEM((CH, 16), jnp.float32),
            pltpu.VMEM((ZR, 16), jnp.float32),
            pltpu.VMEM_SHARED((NP, 16), jnp.float32),
        ],
    )(sidx)


_NB = 10
_BLK = N // _NB


def _tc_pre(x, wfl, wbl, wfr, wbr):
    def body(x_ref, wfl_ref, wbl_ref, wfr_ref, wbr_ref, y_ref, r_ref):
        xb = x_ref[...]
        y_ref[0] = _dot(xb, wfl_ref[...])
        y_ref[1] = _dot(xb, wbl_ref[...])
        r_ref[0] = _dot(xb, wfr_ref[...])
        r_ref[1] = _dot(xb, wbr_ref[...])

    w_spec = pl.BlockSpec((D, D), lambda i: (0, 0))
    return pl.pallas_call(
        body,
        grid=(_NB,),
        in_specs=[pl.BlockSpec((_BLK, D), lambda i: (i, 0)),
                  w_spec, w_spec, w_spec, w_spec],
        out_specs=[pl.BlockSpec((2, _BLK, D), lambda i: (0, i, 0)),
                   pl.BlockSpec((2, _BLK, D), lambda i: (0, i, 0))],
        out_shape=[jax.ShapeDtypeStruct((2, N, D), jnp.float32),
                   jax.ShapeDtypeStruct((2, N, D), jnp.float32)],
    )(x, wfl, wbl, wfr, wbr)


def _tc_stats(agg, cnt, rtab, bf, bb):
    def body(agg_ref, cnt_ref, rtab_ref, bf_ref, bb_ref, h_ref, s_ref, q_ref):
        i = pl.program_id(0)
        cin = jnp.maximum(cnt_ref[0, :, 0:1], 1.0)
        cout = jnp.maximum(cnt_ref[1, :, 0:1], 1.0)
        hf = agg_ref[0] / cin + bf_ref[...] + rtab_ref[0]
        hb = agg_ref[1] / cout + bb_ref[...] + rtab_ref[1]
        h = jnp.concatenate([hf, hb], axis=1)
        h_ref[...] = h

        @pl.when(i == 0)
        def _():
            s_ref[...] = jnp.zeros_like(s_ref)
            q_ref[...] = jnp.zeros_like(q_ref)

        s_ref[...] += jnp.sum(h, axis=0, keepdims=True)
        q_ref[...] += jnp.sum(h * h, axis=0, keepdims=True)

    b_spec = pl.BlockSpec((1, D), lambda i: (0, 0))
    stat_spec = pl.BlockSpec((1, HID), lambda i: (0, 0))
    return pl.pallas_call(
        body,
        grid=(_NB,),
        in_specs=[pl.BlockSpec((2, _BLK, D), lambda i: (0, i, 0)),
                  pl.BlockSpec((2, _BLK, D), lambda i: (0, i, 0)),
                  pl.BlockSpec((2, _BLK, D), lambda i: (0, i, 0)),
                  b_spec, b_spec],
        out_specs=[pl.BlockSpec((_BLK, HID), lambda i: (i, 0)),
                   stat_spec, stat_spec],
        out_shape=[jax.ShapeDtypeStruct((N, HID), jnp.float32),
                   jax.ShapeDtypeStruct((1, HID), jnp.float32),
                   jax.ShapeDtypeStruct((1, HID), jnp.float32)],
    )(agg, cnt, rtab, bf, bb)


def _bn_relu(h, s, q, g, b):
    m = s / N
    v = q / N - m * m
    return jnp.maximum((h - m) * lax.rsqrt(v + EPS) * g + b, 0.0)


def _tc_apply_mid(h, s, q, g, b, wfl, wbl, wfr, wbr):
    def body(h_ref, s_ref, q_ref, g_ref, b_ref,
             wfl_ref, wbl_ref, wfr_ref, wbr_ref, y2_ref, r2_ref):
        hn = _bn_relu(h_ref[...], s_ref[...], q_ref[...],
                      g_ref[...], b_ref[...])
        y2_ref[0] = _dot(hn, wfl_ref[...])
        y2_ref[1] = _dot(hn, wbl_ref[...])
        r2_ref[0] = _dot(hn, wfr_ref[...])
        r2_ref[1] = _dot(hn, wbr_ref[...])

    stat_spec = pl.BlockSpec((1, HID), lambda i: (0, 0))
    w_spec = pl.BlockSpec((HID, D), lambda i: (0, 0))
    return pl.pallas_call(
        body,
        grid=(_NB,),
        in_specs=[pl.BlockSpec((_BLK, HID), lambda i: (i, 0)),
                  stat_spec, stat_spec, stat_spec, stat_spec,
                  w_spec, w_spec, w_spec, w_spec],
        out_specs=[pl.BlockSpec((2, _BLK, D), lambda i: (0, i, 0)),
                   pl.BlockSpec((2, _BLK, D), lambda i: (0, i, 0))],
        out_shape=[jax.ShapeDtypeStruct((2, N, D), jnp.float32),
                   jax.ShapeDtypeStruct((2, N, D), jnp.float32)],
    )(h, s, q, g, b, wfl, wbl, wfr, wbr)


def _tc_apply_post(h, s, q, g, b, wp, bp):
    def body(h_ref, s_ref, q_ref, g_ref, b_ref, wp_ref, bp_ref,
             embs_ref, genc_ref):
        i = pl.program_id(0)
        embs = _bn_relu(h_ref[...], s_ref[...], q_ref[...],
                        g_ref[...], b_ref[...])
        embs_ref[...] = embs
        proj = jnp.maximum(_dot(embs, wp_ref[...]) + bp_ref[...], 0.0)
        pmax = jnp.max(proj, axis=0, keepdims=True)

        @pl.when(i == 0)
        def _():
            genc_ref[...] = jnp.zeros_like(genc_ref)

        genc_ref[...] = jnp.maximum(genc_ref[...], pmax)

    stat_spec = pl.BlockSpec((1, HID), lambda i: (0, 0))
    return pl.pallas_call(
        body,
        grid=(_NB,),
        in_specs=[pl.BlockSpec((_BLK, HID), lambda i: (i, 0)),
                  stat_spec, stat_spec, stat_spec, stat_spec,
                  pl.BlockSpec((HID, HID), lambda i: (0, 0)),
                  stat_spec],
        out_specs=[pl.BlockSpec((_BLK, HID), lambda i: (i, 0)),
                   stat_spec],
        out_shape=[jax.ShapeDtypeStruct((N, HID), jnp.float32),
                   jax.ShapeDtypeStruct((1, HID), jnp.float32)],
    )(h, s, q, g, b, wp, bp)


def kernel(x, edge_index, W1f_l, b1f, W1f_r, W1b_l, b1b, W1b_r, bn1_g, bn1_b,
           W2f_l, b2f, W2f_r, W2b_l, b2b, W2b_r, bn2_g, bn2_b, Wp, bp):
    src = edge_index[0]
    dst = edge_index[1]
    pad = EPC - E
    zpad = jnp.zeros((pad,), jnp.int32)
    tpad = jnp.full((pad,), N, jnp.int32)
    gidx = jnp.stack([jnp.concatenate([src, zpad]),
                      jnp.concatenate([dst + N, zpad])])
    sidx = jnp.stack([jnp.concatenate([dst, tpad]),
                      jnp.concatenate([src, tpad])])
    gidx = gidx.reshape(NC, NS, NBLK, SB, CH)
    sidx = sidx.reshape(NC, NS, NBLK, SB, CH)

    ones_tab = jnp.ones((NC * N, D), jnp.float32)
    cnt = _aggregate(ones_tab, gidx, sidx)
    ytab1, rtab1 = _tc_pre(x, W1f_l, W1b_l, W1f_r, W1b_r)
    agg1 = _aggregate(ytab1.reshape(NC * N, D), gidx, sidx)
    h1, s1, q1 = _tc_stats(agg1, cnt, rtab1,
                           b1f.reshape(1, D), b1b.reshape(1, D))
    ytab2, rtab2 = _tc_apply_mid(h1, s1, q1,
                                 bn1_g.reshape(1, HID), bn1_b.reshape(1, HID),
                                 W2f_l, W2b_l, W2f_r, W2b_r)
    agg2 = _aggregate(ytab2.reshape(NC * N, D), gidx, sidx)
    h2, s2, q2 = _tc_stats(agg2, cnt, rtab2,
                           b2f.reshape(1, D), b2b.reshape(1, D))
    node_embs, genc = _tc_apply_post(h2, s2, q2,
                                     bn2_g.reshape(1, HID),
                                     bn2_b.reshape(1, HID),
                                     Wp, bp.reshape(1, HID))
    return (node_embs, genc.reshape(HID))
